# baseline (device time: 43278 ns/iter reference)
import jax
import jax.numpy as jnp
from jax import lax
from jax.experimental import pallas as pl
from jax.experimental.pallas import tpu as pltpu

N_DEV = 4
N_LAYERS = 3
N_PHASES = 2 * N_LAYERS - 1
BF = jnp.bfloat16

SEND_ORDER = (2, 1, 3)
RECV_ORDER = (1, 3, 2)


def kernel(x, Win0, Wout0, Win1, Wout1, Win2, Wout2):
    b, d = x.shape
    rows_per = b // N_DEV
    cols_per = d // N_DEV

    def body(x_ref, win0_ref, wout0_ref, win1_ref, wout1_ref, win2_ref,
             wout2_ref, out_ref, x_buf, part_ref, red_ref, rs_ref,
             part2_ref, rs2_ref, send_sems, recv_sems):
        my = lax.axis_index("i")

        barrier_sem = pltpu.get_barrier_semaphore()
        for k in range(1, N_DEV):
            pl.semaphore_signal(barrier_sem, inc=1,
                                device_id=((my + k) % N_DEV,),
                                device_id_type=pl.DeviceIdType.MESH)
        pl.semaphore_wait(barrier_sem, N_DEV - 1)

        wins = [win0_ref, win1_ref, win2_ref]
        wouts = [wout0_ref, wout1_ref, wout2_ref]

        h = jnp.maximum(
            jnp.dot(x_ref[:, :].astype(BF), win0_ref[:, :].astype(BF),
                    preferred_element_type=jnp.float32), 0.0)

        for layer in range(N_LAYERS - 1):
            phase_rs = 2 * layer
            phase_ag = 2 * layer + 1
            hb = h.astype(BF)

            rs_sends = []
            for k in SEND_ORDER:
                r = (my + k) % N_DEV
                s = N_DEV - 1 - k
                part_ref[r, :, :] = jnp.dot(
                    hb,
                    wouts[layer][:, pl.ds(r * cols_per, cols_per)].astype(BF),
                    preferred_element_type=jnp.float32)
                rdma = pltpu.make_async_remote_copy(
                    src_ref=part_ref.at[r],
                    dst_ref=rs_ref.at[s],
                    send_sem=send_sems.at[phase_rs, s],
                    recv_sem=recv_sems.at[phase_rs, s],
                    device_id=(r,),
                    device_id_type=pl.DeviceIdType.MESH,
                )
                rdma.start()
                rs_sends.append(rdma)
            p_my = jnp.dot(
                hb,
                wouts[layer][:, pl.ds(my * cols_per, cols_per)].astype(BF),
                preferred_element_type=jnp.float32)
            for rdma in rs_sends:
                rdma.wait()

            reduced = p_my + rs_ref[0] + rs_ref[1] + rs_ref[2]
            red_ref[:, :] = reduced
            ag_sends = []
            for k in SEND_ORDER:
                r = (my + k) % N_DEV
                s = N_DEV - 1 - k
                rdma = pltpu.make_async_remote_copy(
                    src_ref=red_ref,
                    dst_ref=x_buf.at[my],
                    send_sem=send_sems.at[phase_ag, s],
                    recv_sem=recv_sems.at[phase_ag, s],
                    device_id=(r,),
                    device_id_type=pl.DeviceIdType.MESH,
                )
                rdma.start()
                ag_sends.append(rdma)

            wnext = wins[layer + 1]
            hacc = jnp.dot(
                reduced.astype(BF),
                wnext[pl.ds(my * cols_per, cols_per), :].astype(BF),
                preferred_element_type=jnp.float32)
            for k in RECV_ORDER:
                s = N_DEV - 1 - k
                c = (my - k) % N_DEV
                recv = pltpu.make_async_remote_copy(
                    src_ref=red_ref,
                    dst_ref=x_buf.at[c],
                    send_sem=send_sems.at[phase_ag, s],
                    recv_sem=recv_sems.at[phase_ag, s],
                    device_id=(c,),
                    device_id_type=pl.DeviceIdType.MESH,
                )
                recv.wait_recv()
                hacc = hacc + jnp.dot(
                    x_buf[c].astype(BF),
                    wnext[pl.ds(c * cols_per, cols_per), :].astype(BF),
                    preferred_element_type=jnp.float32)
            for rdma in ag_sends:
                rdma.wait_send()
            h = jnp.maximum(hacc, 0.0)

        hb = h.astype(BF)
        part2_ref[:, :] = jnp.dot(hb, wout2_ref[:, :].astype(BF),
                                  preferred_element_type=jnp.float32)
        phase = N_PHASES - 1
        sends = []
        for k in SEND_ORDER:
            r = (my + k) % N_DEV
            s = N_DEV - 1 - k
            rdma = pltpu.make_async_remote_copy(
                src_ref=part2_ref.at[pl.ds(r * rows_per, rows_per), :],
                dst_ref=rs2_ref.at[s],
                send_sem=send_sems.at[phase, s],
                recv_sem=recv_sems.at[phase, s],
                device_id=(r,),
                device_id_type=pl.DeviceIdType.MESH,
            )
            rdma.start()
            sends.append(rdma)
        for rdma in sends:
            rdma.wait()
        out_ref[:, :] = (part2_ref[pl.ds(my * rows_per, rows_per), :]
                         + rs2_ref[0] + rs2_ref[1] + rs2_ref[2])

    return pl.pallas_call(
        body,
        out_shape=jax.ShapeDtypeStruct((rows_per, d), jnp.float32),
        in_specs=[pl.BlockSpec(memory_space=pltpu.VMEM)] * 7,
        out_specs=pl.BlockSpec(memory_space=pltpu.VMEM),
        scratch_shapes=[
            pltpu.VMEM((N_DEV, b, cols_per), jnp.float32),
            pltpu.VMEM((N_DEV, b, cols_per), jnp.float32),
            pltpu.VMEM((b, cols_per), jnp.float32),
            pltpu.VMEM((N_DEV - 1, b, cols_per), jnp.float32),
            pltpu.VMEM((b, d), jnp.float32),
            pltpu.VMEM((N_DEV - 1, rows_per, d), jnp.float32),
            pltpu.SemaphoreType.DMA((N_PHASES, N_DEV - 1)),
            pltpu.SemaphoreType.DMA((N_PHASES, N_DEV - 1)),
        ],
        compiler_params=pltpu.CompilerParams(
            collective_id=0,
            vmem_limit_bytes=100 * 1024 * 1024,
        ),
    )(x, Win0, Wout0, Win1, Wout1, Win2, Wout2)
